# baseline (speedup 1.0000x reference)
import jax
import jax.numpy as jnp
from jax.experimental import pallas as pl

NUM_EMB = 1024
DIM = 256
LAYERS = 4
CC = 0.25

BLK = 4608
HALF = BLK // 2


def _rvq_block(x_ref, w_ref, w2_ref, wsq_ref, q_ref, idx_ref, loss_ref):
    @pl.when(pl.program_id(0) == 0)
    def _init():
        loss_ref[...] = jnp.zeros((1, 1), jnp.float32)

    LT = 128
    NT = NUM_EMB // LT
    iota_f = jax.lax.broadcasted_iota(
        jnp.int32, (HALF, NUM_EMB), 1).astype(jnp.float32)
    lane_f = jax.lax.broadcasted_iota(
        jnp.int32, (HALF, LT), 1).astype(jnp.float32)

    def vq_layer(residual, w, w2, b):
        a = jnp.sum(residual * residual, axis=1, keepdims=True)
        m2 = jax.lax.dot_general(
            residual, w2,
            dimension_numbers=(((1,), (1,)), ((), ())),
            preferred_element_type=jnp.float32,
        )
        run_v = (a + b[:, 0:LT]) - m2[:, 0:LT]
        run_i = lane_f
        for t in range(1, NT):
            dist_t = (a + b[:, t * LT:(t + 1) * LT]) - m2[:, t * LT:(t + 1) * LT]
            take = dist_t < run_v
            run_v = jnp.minimum(run_v, dist_t)
            run_i = jnp.where(take, lane_f + jnp.float32(t * LT), run_i)
        mn = jnp.min(run_v, axis=1, keepdims=True)
        idx_f = jnp.min(jnp.where(run_v == mn, run_i, jnp.float32(NUM_EMB)),
                        axis=1, keepdims=True)
        onehot = (iota_f == idx_f).astype(jnp.float32)
        q = jax.lax.dot_general(
            onehot, w,
            dimension_numbers=(((1,), (0,)), ((), ())),
            preferred_element_type=jnp.float32,
        )
        return q, idx_f, a

    NS = BLK // HALF
    sls = [pl.ds(i * HALF, HALF) for i in range(NS)]
    res = [x_ref[s, :] for s in sls]
    q_acc = [jnp.zeros_like(r) for r in res]
    loss_cols = [jnp.zeros((HALF, 1), jnp.float32) for _ in range(NS)]
    for l in range(LAYERS):
        for h in range(NS):
            q, idx_f, a = vq_layer(res[h], w_ref[l], w2_ref[l], wsq_ref[l])
            if l > 0:
                loss_cols[h] = loss_cols[h] + a
            idx_ref[sls[h], l:l + 1] = idx_f.astype(jnp.int32)
            res[h] = res[h] - q
            q_acc[h] = q_acc[h] + q

    loss_tot = jnp.zeros((1, 1), jnp.float32)
    for h in range(NS):
        lc = loss_cols[h] + jnp.sum(res[h] * res[h], axis=1, keepdims=True)
        q_ref[sls[h], :] = q_acc[h]
        loss_tot = loss_tot + jnp.sum(lc).reshape(1, 1)
    loss_ref[...] += loss_tot


def kernel(x, W):
    nb = (16 * 576) // BLK
    flat = x.reshape(-1, DIM)
    wsq = jnp.stack([jnp.sum(W[l] ** 2, axis=1) for l in range(LAYERS)])
    wsq = wsq.reshape(LAYERS, 1, NUM_EMB)

    q_flat, idx_blk, loss_sum = pl.pallas_call(
        _rvq_block,
        grid=(nb,),
        in_specs=[
            pl.BlockSpec((BLK, DIM), lambda i: (i, 0)),
            pl.BlockSpec((LAYERS, NUM_EMB, DIM), lambda i: (0, 0, 0)),
            pl.BlockSpec((LAYERS, NUM_EMB, DIM), lambda i: (0, 0, 0)),
            pl.BlockSpec((LAYERS, 1, NUM_EMB), lambda i: (0, 0, 0)),
        ],
        out_specs=[
            pl.BlockSpec((BLK, DIM), lambda i: (i, 0)),
            pl.BlockSpec((BLK, 8), lambda i: (i, 0)),
            pl.BlockSpec((1, 1), lambda i: (0, 0)),
        ],
        out_shape=[
            jax.ShapeDtypeStruct((nb * BLK, DIM), jnp.float32),
            jax.ShapeDtypeStruct((nb * BLK, 8), jnp.int32),
            jax.ShapeDtypeStruct((1, 1), jnp.float32),
        ],
    )(flat, W, 2.0 * W, wsq)

    quantized_out = q_flat.reshape(x.shape)
    all_indices = idx_blk[:, :LAYERS].reshape(x.shape[0], x.shape[1], LAYERS)
    all_losses = (1.0 + CC) * (loss_sum[0, 0] / jnp.float32(flat.shape[0] * DIM))
    return (quantized_out, all_losses, all_indices)


# speedup vs baseline: 1.0274x; 1.0274x over previous
import jax
import jax.numpy as jnp
from jax.experimental import pallas as pl

NUM_EMB = 1024
DIM = 256
LAYERS = 4
CC = 0.25

BLK = 2304
HALF = BLK // 2


def _rvq_block(x_ref, w2_ref, wsq_ref, q_ref, idx_ref, loss_ref):
    @pl.when(pl.program_id(0) == 0)
    def _init():
        loss_ref[...] = jnp.zeros((1, 1), jnp.float32)

    LT = 128
    NT = NUM_EMB // LT
    iota_f = jax.lax.broadcasted_iota(
        jnp.int32, (HALF, NUM_EMB), 1).astype(jnp.float32)
    lane_f = jax.lax.broadcasted_iota(
        jnp.int32, (HALF, LT), 1).astype(jnp.float32)

    def vq_layer(residual, w2, b):
        a = jnp.sum(residual * residual, axis=1, keepdims=True)
        m2 = jax.lax.dot_general(
            residual, w2,
            dimension_numbers=(((1,), (1,)), ((), ())),
            preferred_element_type=jnp.float32,
        )
        run_v = (a + b[:, 0:LT]) - m2[:, 0:LT]
        run_i = lane_f
        for t in range(1, NT):
            dist_t = (a + b[:, t * LT:(t + 1) * LT]) - m2[:, t * LT:(t + 1) * LT]
            take = dist_t < run_v
            run_v = jnp.minimum(run_v, dist_t)
            run_i = jnp.where(take, lane_f + jnp.float32(t * LT), run_i)
        mn = jnp.min(run_v, axis=1, keepdims=True)
        idx_f = jnp.min(jnp.where(run_v == mn, run_i, jnp.float32(NUM_EMB)),
                        axis=1, keepdims=True)
        onehot = (iota_f == idx_f).astype(jnp.float32)
        q = 0.5 * jax.lax.dot_general(
            onehot, w2,
            dimension_numbers=(((1,), (0,)), ((), ())),
            preferred_element_type=jnp.float32,
        )
        return q, idx_f, a

    NS = BLK // HALF
    sls = [pl.ds(i * HALF, HALF) for i in range(NS)]
    res = [x_ref[s, :] for s in sls]
    q_acc = [jnp.zeros_like(r) for r in res]
    loss_cols = [jnp.zeros((HALF, 1), jnp.float32) for _ in range(NS)]
    for l in range(LAYERS):
        for h in range(NS):
            q, idx_f, a = vq_layer(res[h], w2_ref[l], wsq_ref[l])
            if l > 0:
                loss_cols[h] = loss_cols[h] + a
            idx_ref[sls[h], l:l + 1] = idx_f.astype(jnp.int32)
            res[h] = res[h] - q
            q_acc[h] = q_acc[h] + q

    loss_tot = jnp.zeros((1, 1), jnp.float32)
    for h in range(NS):
        lc = loss_cols[h] + jnp.sum(res[h] * res[h], axis=1, keepdims=True)
        q_ref[sls[h], :] = q_acc[h]
        loss_tot = loss_tot + jnp.sum(lc).reshape(1, 1)
    loss_ref[...] += loss_tot


def kernel(x, W):
    nb = (16 * 576) // BLK
    flat = x.reshape(-1, DIM)
    wsq = jnp.stack([jnp.sum(W[l] ** 2, axis=1) for l in range(LAYERS)])
    wsq = wsq.reshape(LAYERS, 1, NUM_EMB)

    q_flat, idx_blk, loss_sum = pl.pallas_call(
        _rvq_block,
        grid=(nb,),
        in_specs=[
            pl.BlockSpec((BLK, DIM), lambda i: (i, 0)),
            pl.BlockSpec((LAYERS, NUM_EMB, DIM), lambda i: (0, 0, 0)),
            pl.BlockSpec((LAYERS, 1, NUM_EMB), lambda i: (0, 0, 0)),
        ],
        out_specs=[
            pl.BlockSpec((BLK, DIM), lambda i: (i, 0)),
            pl.BlockSpec((BLK, 8), lambda i: (i, 0)),
            pl.BlockSpec((1, 1), lambda i: (0, 0)),
        ],
        out_shape=[
            jax.ShapeDtypeStruct((nb * BLK, DIM), jnp.float32),
            jax.ShapeDtypeStruct((nb * BLK, 8), jnp.int32),
            jax.ShapeDtypeStruct((1, 1), jnp.float32),
        ],
    )(flat, 2.0 * W, wsq)

    quantized_out = q_flat.reshape(x.shape)
    all_indices = idx_blk[:, :LAYERS].reshape(x.shape[0], x.shape[1], LAYERS)
    all_losses = (1.0 + CC) * (loss_sum[0, 0] / jnp.float32(flat.shape[0] * DIM))
    return (quantized_out, all_losses, all_indices)
